# baseline (speedup 1.0000x reference)
import functools

import jax
import jax.numpy as jnp
from jax import lax
from jax.experimental import pallas as pl
from jax.experimental.pallas import tpu as pltpu
from jax.experimental.pallas import tpu_sc as plsc

NC = 2
NS = 16
CHUNK = 128
G = 16
DCOL = 4


def _make_sc_agg(feat, p_pad, n_acc):
  n_groups = p_pad // G
  pg = p_pad + G
  rows_per_tile = n_acc // NS
  mesh = plsc.VectorSubcoreMesh(core_axis_name="c", subcore_axis_name="s")

  @functools.partial(
      pl.kernel,
      out_type=jax.ShapeDtypeStruct((NC, n_acc, feat), jnp.float32),
      mesh=mesh,
      scratch_types=[
          pltpu.VMEM((2, G, CHUNK), jnp.int32),
          pltpu.VMEM((2, G, CHUNK), jnp.int32),
          pltpu.VMEM((CHUNK, feat), jnp.float32),
          pltpu.VMEM((CHUNK, feat), jnp.float32),
          pltpu.VMEM_SHARED((n_acc, feat), jnp.float32),
          pltpu.SemaphoreType.DMA,
          pltpu.SemaphoreType.DMA,
          pltpu.SemaphoreType.DMA,
      ],
      compiler_params=pltpu.CompilerParams(needs_layout_passes=False))
  def k(table_hbm, src4_hbm, dst4_hbm, zeros_hbm, out_hbm,
        sbuf, dbuf, b0, b1, acc_sh, sg0, sg1, si):
    c = lax.axis_index("c")
    s = lax.axis_index("s")
    bufs = (b0, b1)
    sems = (sg0, sg1)
    row0 = s * rows_per_tile

    pltpu.sync_copy(zeros_hbm, acc_sh.at[pl.ds(row0, rows_per_tile)])
    plsc.subcore_barrier()

    pltpu.sync_copy(src4_hbm.at[c, s, pl.ds(0, G)], sbuf.at[0])
    pltpu.sync_copy(dst4_hbm.at[c, s, pl.ds(0, G)], dbuf.at[0])
    pltpu.async_copy(table_hbm.at[sbuf.at[0, 0]], b0, sg0)

    def group(g, carry):
      par = g & 1
      nxt = 1 - par
      pltpu.async_copy(src4_hbm.at[c, s, pl.ds((g + 1) * G, G)],
                       sbuf.at[nxt], si)
      pltpu.async_copy(dst4_hbm.at[c, s, pl.ds((g + 1) * G, G)],
                       dbuf.at[nxt], si)
      for j in range(G):
        pb = j & 1
        nb = (j + 1) & 1
        if j + 1 < G:
          nxt_idx = sbuf.at[par, j + 1]
        else:
          pltpu.make_async_copy(src4_hbm.at[c, s, pl.ds(0, G)],
                                sbuf.at[par], si).wait()
          pltpu.make_async_copy(dst4_hbm.at[c, s, pl.ds(0, G)],
                                dbuf.at[par], si).wait()
          nxt_idx = sbuf.at[nxt, 0]
        pltpu.async_copy(table_hbm.at[nxt_idx], bufs[nb], sems[nb])
        pltpu.make_async_copy(table_hbm.at[pl.ds(0, CHUNK)],
                              bufs[pb], sems[pb]).wait()
        pltpu.sync_copy(bufs[pb], acc_sh.at[dbuf.at[par, j]], add=True)
      return carry

    lax.fori_loop(0, n_groups, group, 0)
    pltpu.make_async_copy(table_hbm.at[pl.ds(0, CHUNK)], b0, sg0).wait()
    plsc.subcore_barrier()

    pltpu.sync_copy(acc_sh.at[pl.ds(row0, rows_per_tile)],
                    out_hbm.at[c, pl.ds(row0, rows_per_tile)])

  return k


def _make_sc_deg(pg_words, p_pad, n_acc):
  mesh = plsc.VectorSubcoreMesh(core_axis_name="c", subcore_axis_name="s")

  @functools.partial(
      pl.kernel,
      out_type=jax.ShapeDtypeStruct((NC, NS, n_acc), jnp.float32),
      mesh=mesh,
      scratch_types=[
          pltpu.VMEM((pg_words,), jnp.int32),
          pltpu.VMEM((DCOL * n_acc,), jnp.float32),
      ],
      compiler_params=pltpu.CompilerParams(needs_layout_passes=False))
  def k(dstf_hbm, zdeg_hbm, odeg_hbm, dstv, deg_v):
    c = lax.axis_index("c")
    s = lax.axis_index("s")
    pltpu.sync_copy(zdeg_hbm, deg_v)
    pltpu.sync_copy(dstf_hbm.at[c, s], dstv)
    lane = lax.iota(jnp.int32, 16)
    colbase = (lane & (DCOL - 1)) * n_acc
    ones16 = jnp.full((16,), 1.0, jnp.float32)
    gmasks = [(lane // DCOL) == g for g in range(16 // DCOL)]

    def body(t, carry):
      for q in range(8):
        dv = dstv[pl.ds(t * 128 + q * 16, 16)]
        idxv = dv + colbase
        for m in gmasks:
          plsc.addupdate_scatter(deg_v, [idxv], ones16, mask=m)
      return carry

    lax.fori_loop(0, p_pad, body, 0)

    def red(i, carry):
      for u in range(4):
        o = (i * 4 + u) * 16
        v = deg_v[pl.ds(o, 16)]
        for d in range(1, DCOL):
          v += deg_v[pl.ds(d * n_acc + o, 16)]
        deg_v[pl.ds(o, 16)] = v
      return carry

    lax.fori_loop(0, n_acc // 64, red, 0)
    pltpu.sync_copy(deg_v.at[pl.ds(0, n_acc)], odeg_hbm.at[c, s])

  return k


def _tc_layer1(parts, degp, x_pad, wl, wr, b, n_acc, in_dim, h_dim):
  BN = 512
  feat = parts.shape[2]
  hh = h_dim // 2

  def body(parts_ref, degp_ref, x_ref, wl_ref, wr_ref, b_ref,
           h1_ref, invd_ref):
    deg = jnp.sum(degp_ref[...], axis=(0, 1))
    invd = 1.0 / jnp.maximum(deg, 1.0)
    p = parts_ref[0] + parts_ref[1]
    mean = p * invd[:, None]
    h = jnp.dot(mean, wl_ref[...], preferred_element_type=jnp.float32)
    h += jnp.dot(x_ref[...], wr_ref[...], preferred_element_type=jnp.float32)
    h = jnp.maximum(h + b_ref[...], 0.0)
    h1_ref[0] = h[:, :hh]
    h1_ref[1] = h[:, hh:]
    invd_ref[...] = invd

  return pl.pallas_call(
      body,
      grid=(n_acc // BN,),
      in_specs=[
          pl.BlockSpec((NC, BN, feat), lambda i: (0, i, 0)),
          pl.BlockSpec((NC, NS, BN), lambda i: (0, 0, i)),
          pl.BlockSpec((BN, in_dim), lambda i: (i, 0)),
          pl.BlockSpec((in_dim, h_dim), lambda i: (0, 0)),
          pl.BlockSpec((in_dim, h_dim), lambda i: (0, 0)),
          pl.BlockSpec((h_dim,), lambda i: (0,)),
      ],
      out_specs=[
          pl.BlockSpec((NC, BN, hh), lambda i: (0, i, 0)),
          pl.BlockSpec((BN,), lambda i: (i,)),
      ],
      out_shape=[
          jax.ShapeDtypeStruct((NC, n_acc, hh), jnp.float32),
          jax.ShapeDtypeStruct((n_acc,), jnp.float32),
      ],
  )(parts, degp, x_pad, wl, wr, b)


def _tc_layer2(parts2, h1, invd, wl2, wr2, b2, wc_pad, bc_pad, n_acc, h_dim):
  BN = 512
  hh = h_dim // 2
  oc = wc_pad.shape[1]

  def body(p2_ref, h1_ref, invd_ref, wl_ref, wr_ref, b_ref, wc_ref, bc_ref,
           out_ref):
    agg = jnp.concatenate([p2_ref[0], p2_ref[1]], axis=1)
    mean = agg * invd_ref[...][:, None]
    hp = jnp.concatenate([h1_ref[0], h1_ref[1]], axis=1)
    h = jnp.dot(mean, wl_ref[...], preferred_element_type=jnp.float32)
    h += jnp.dot(hp, wr_ref[...], preferred_element_type=jnp.float32)
    h = jnp.maximum(h + b_ref[...], 0.0)
    out_ref[...] = (
        jnp.dot(h, wc_ref[...], preferred_element_type=jnp.float32)
        + bc_ref[...])

  return pl.pallas_call(
      body,
      grid=(n_acc // BN,),
      in_specs=[
          pl.BlockSpec((NC, BN, hh), lambda i: (0, i, 0)),
          pl.BlockSpec((NC, BN, hh), lambda i: (0, i, 0)),
          pl.BlockSpec((BN,), lambda i: (i,)),
          pl.BlockSpec((h_dim, h_dim), lambda i: (0, 0)),
          pl.BlockSpec((h_dim, h_dim), lambda i: (0, 0)),
          pl.BlockSpec((h_dim,), lambda i: (0,)),
          pl.BlockSpec((h_dim, oc), lambda i: (0, 0)),
          pl.BlockSpec((oc,), lambda i: (0,)),
      ],
      out_specs=pl.BlockSpec((BN, oc), lambda i: (i, 0)),
      out_shape=jax.ShapeDtypeStruct((n_acc, oc), jnp.float32),
  )(parts2, h1, invd, wl2, wr2, b2, wc_pad, bc_pad)


def _chunked_idx(vals, fill, n_rows, p, p_pad):
  cap = n_rows * p * CHUNK
  v = jnp.concatenate(
      [vals, jnp.full((cap - vals.shape[0],), fill, vals.dtype)])
  v = v.reshape(n_rows, p, CHUNK)
  padc = jnp.full((n_rows, p_pad + G - p, CHUNK), fill, vals.dtype)
  return jnp.concatenate([v, padc], axis=1)


def kernel(x, edge_index, W_l1, W_r1, b1, W_l2, W_r2, b2, Wc, bc):
  n, in_dim = x.shape
  e = edge_index.shape[1]
  h_dim = W_l1.shape[1]
  out_dim = Wc.shape[1]
  hh = h_dim // 2

  n_acc = -(-(n + 1) // (NS * 128)) * (NS * 128)
  rows_per_tile = n_acc // NS

  src = edge_index[0]
  dst = edge_index[1]

  p1 = -(-e // (NC * NS * CHUNK))
  p1_pad = -(-p1 // G) * G
  src1 = _chunked_idx(src, 0, NC * NS, p1, p1_pad).reshape(
      NC, NS, p1_pad + G, CHUNK)
  dst1 = _chunked_idx(dst, n, NC * NS, p1, p1_pad).reshape(
      NC, NS, p1_pad + G, CHUNK)
  zeros1 = jnp.zeros((rows_per_tile, in_dim), jnp.float32)

  agg1 = _make_sc_agg(in_dim, p1_pad, n_acc)
  parts1 = agg1(x, src1, dst1, zeros1)

  zdeg = jnp.zeros((DCOL * n_acc,), jnp.float32)
  dst1f = dst1.reshape(NC, NS, (p1_pad + G) * CHUNK)
  deg_k = _make_sc_deg((p1_pad + G) * CHUNK, p1_pad, n_acc)
  degp = deg_k(dst1f, zdeg)

  x_pad = jnp.concatenate(
      [x, jnp.zeros((n_acc - n, in_dim), jnp.float32)], axis=0)
  h1, invd = _tc_layer1(parts1, degp, x_pad, W_l1, W_r1, b1,
                        n_acc, in_dim, h_dim)

  table2 = h1.reshape(NC * n_acc, hh)
  p2 = -(-e // (NS * CHUNK))
  p2_pad = -(-p2 // G) * G
  src2a = _chunked_idx(src, 0, NS, p2, p2_pad)
  src2 = jnp.stack([src2a, src2a + n_acc])
  dst2a = _chunked_idx(dst, n, NS, p2, p2_pad)
  dst2 = jnp.stack([dst2a, dst2a])
  zeros2 = jnp.zeros((rows_per_tile, hh), jnp.float32)

  agg2 = _make_sc_agg(hh, p2_pad, n_acc)
  parts2 = agg2(table2, src2, dst2, zeros2)

  oc = 128
  wc_pad = jnp.zeros((h_dim, oc), jnp.float32).at[:, :out_dim].set(Wc)
  bc_pad = jnp.zeros((oc,), jnp.float32).at[:out_dim].set(bc)
  out = _tc_layer2(parts2, h1, invd, W_l2, W_r2, b2, wc_pad, bc_pad,
                   n_acc, h_dim)
  return out[:n, :out_dim]


# speedup vs baseline: 3.0267x; 3.0267x over previous
import functools

import jax
import jax.numpy as jnp
from jax import lax
from jax.experimental import pallas as pl
from jax.experimental.pallas import tpu as pltpu
from jax.experimental.pallas import tpu_sc as plsc

NC = 2
NS = 16
CHUNK = 128
G = 16
DCOL = 4


def _make_sc_agg(feat, p_pad, n_acc):
  n_groups = p_pad // G
  pg = p_pad + G
  rows_per_tile = n_acc // NS
  mesh = plsc.VectorSubcoreMesh(core_axis_name="c", subcore_axis_name="s")

  @functools.partial(
      pl.kernel,
      out_type=jax.ShapeDtypeStruct((NC, n_acc, feat), jnp.float32),
      mesh=mesh,
      scratch_types=[
          pltpu.VMEM((2, G, CHUNK), jnp.int32),
          pltpu.VMEM((2, G, CHUNK), jnp.int32),
          pltpu.VMEM((CHUNK, feat), jnp.float32),
          pltpu.VMEM((CHUNK, feat), jnp.float32),
          pltpu.VMEM_SHARED((n_acc, feat), jnp.float32),
          pltpu.SemaphoreType.DMA,
          pltpu.SemaphoreType.DMA,
          pltpu.SemaphoreType.DMA,
      ],
      compiler_params=pltpu.CompilerParams(needs_layout_passes=False))
  def k(table_hbm, src4_hbm, dst4_hbm, zeros_hbm, out_hbm,
        sbuf, dbuf, b0, b1, acc_sh, sg0, sg1, si):
    c = lax.axis_index("c")
    s = lax.axis_index("s")
    bufs = (b0, b1)
    sems = (sg0, sg1)
    row0 = s * rows_per_tile

    pltpu.sync_copy(zeros_hbm, acc_sh.at[pl.ds(row0, rows_per_tile)])
    plsc.subcore_barrier()

    pltpu.sync_copy(src4_hbm.at[c, s, pl.ds(0, G)], sbuf.at[0])
    pltpu.sync_copy(dst4_hbm.at[c, s, pl.ds(0, G)], dbuf.at[0])
    pltpu.async_copy(table_hbm.at[sbuf.at[0, 0]], b0, sg0)

    def group(g, carry):
      par = g & 1
      nxt = 1 - par
      pltpu.async_copy(src4_hbm.at[c, s, pl.ds((g + 1) * G, G)],
                       sbuf.at[nxt], si)
      pltpu.async_copy(dst4_hbm.at[c, s, pl.ds((g + 1) * G, G)],
                       dbuf.at[nxt], si)
      for j in range(G):
        pb = j & 1
        nb = (j + 1) & 1
        if j + 1 < G:
          nxt_idx = sbuf.at[par, j + 1]
        else:
          pltpu.make_async_copy(src4_hbm.at[c, s, pl.ds(0, G)],
                                sbuf.at[par], si).wait()
          pltpu.make_async_copy(dst4_hbm.at[c, s, pl.ds(0, G)],
                                dbuf.at[par], si).wait()
          nxt_idx = sbuf.at[nxt, 0]
        pltpu.async_copy(table_hbm.at[nxt_idx], bufs[nb], sems[nb])
        pltpu.make_async_copy(table_hbm.at[pl.ds(0, CHUNK)],
                              bufs[pb], sems[pb]).wait()
        pltpu.sync_copy(bufs[pb], acc_sh.at[dbuf.at[par, j]], add=True)
      return carry

    lax.fori_loop(0, n_groups, group, 0)
    pltpu.make_async_copy(table_hbm.at[pl.ds(0, CHUNK)], b0, sg0).wait()
    plsc.subcore_barrier()

    pltpu.sync_copy(acc_sh.at[pl.ds(row0, rows_per_tile)],
                    out_hbm.at[c, pl.ds(row0, rows_per_tile)])

  return k


def _make_sc_deg(pg_words, p_pad, n_acc):
  mesh = plsc.VectorSubcoreMesh(core_axis_name="c", subcore_axis_name="s")

  @functools.partial(
      pl.kernel,
      out_type=jax.ShapeDtypeStruct((NC, NS, n_acc), jnp.float32),
      mesh=mesh,
      scratch_types=[
          pltpu.VMEM((pg_words,), jnp.int32),
          pltpu.VMEM((DCOL * n_acc,), jnp.float32),
      ],
      compiler_params=pltpu.CompilerParams(needs_layout_passes=False))
  def k(dstf_hbm, zdeg_hbm, odeg_hbm, dstv, deg_v):
    c = lax.axis_index("c")
    s = lax.axis_index("s")
    pltpu.sync_copy(zdeg_hbm, deg_v)
    pltpu.sync_copy(dstf_hbm.at[c, s], dstv)
    lane = lax.iota(jnp.int32, 16)
    colbase = (lane & (DCOL - 1)) * n_acc
    ones16 = jnp.full((16,), 1.0, jnp.float32)
    gmasks = [(lane // DCOL) == g for g in range(16 // DCOL)]

    def body(t, carry):
      for q in range(8):
        dv = dstv[pl.ds(t * 128 + q * 16, 16)]
        idxv = dv + colbase
        for m in gmasks:
          plsc.addupdate_scatter(deg_v, [idxv], ones16, mask=m)
      return carry

    lax.fori_loop(0, p_pad, body, 0)

    def red(i, carry):
      for u in range(4):
        o = (i * 4 + u) * 16
        v = deg_v[pl.ds(o, 16)]
        for d in range(1, DCOL):
          v += deg_v[pl.ds(d * n_acc + o, 16)]
        deg_v[pl.ds(o, 16)] = v
      return carry

    lax.fori_loop(0, n_acc // 64, red, 0)
    pltpu.sync_copy(deg_v.at[pl.ds(0, n_acc)], odeg_hbm.at[c, s])

  return k


def _tc_layer1(parts, degp, x_pad, wl, wr, b, n_acc, in_dim, h_dim):
  BN = 512
  feat = parts.shape[2]
  hh = h_dim // 2

  def body(parts_ref, degp_ref, x_ref, wl_ref, wr_ref, b_ref,
           h1_ref, invd_ref):
    deg = jnp.sum(degp_ref[...], axis=(0, 1))
    invd = 1.0 / jnp.maximum(deg, 1.0)
    p = parts_ref[0] + parts_ref[1]
    mean = p * invd[:, None]
    h = jnp.dot(mean, wl_ref[...], preferred_element_type=jnp.float32)
    h += jnp.dot(x_ref[...], wr_ref[...], preferred_element_type=jnp.float32)
    h = jnp.maximum(h + b_ref[...], 0.0)
    h1_ref[0] = h[:, :hh]
    h1_ref[1] = h[:, hh:]
    invd_ref[...] = invd

  return pl.pallas_call(
      body,
      grid=(n_acc // BN,),
      in_specs=[
          pl.BlockSpec((NC, BN, feat), lambda i: (0, i, 0)),
          pl.BlockSpec((NC, NS, BN), lambda i: (0, 0, i)),
          pl.BlockSpec((BN, in_dim), lambda i: (i, 0)),
          pl.BlockSpec((in_dim, h_dim), lambda i: (0, 0)),
          pl.BlockSpec((in_dim, h_dim), lambda i: (0, 0)),
          pl.BlockSpec((h_dim,), lambda i: (0,)),
      ],
      out_specs=[
          pl.BlockSpec((NC, BN, hh), lambda i: (0, i, 0)),
          pl.BlockSpec((BN,), lambda i: (i,)),
      ],
      out_shape=[
          jax.ShapeDtypeStruct((NC, n_acc, hh), jnp.float32),
          jax.ShapeDtypeStruct((n_acc,), jnp.float32),
      ],
  )(parts, degp, x_pad, wl, wr, b)


def _tc_layer2(parts2, h1, invd, wl2, wr2, b2, wc_pad, bc_pad, n_acc, h_dim):
  BN = 512
  hh = h_dim // 2
  oc = wc_pad.shape[1]

  def body(p2_ref, h1_ref, invd_ref, wl_ref, wr_ref, b_ref, wc_ref, bc_ref,
           out_ref):
    agg = jnp.concatenate([p2_ref[0], p2_ref[1]], axis=1)
    mean = agg * invd_ref[...][:, None]
    hp = jnp.concatenate([h1_ref[0], h1_ref[1]], axis=1)
    h = jnp.dot(mean, wl_ref[...], preferred_element_type=jnp.float32)
    h += jnp.dot(hp, wr_ref[...], preferred_element_type=jnp.float32)
    h = jnp.maximum(h + b_ref[...], 0.0)
    out_ref[...] = (
        jnp.dot(h, wc_ref[...], preferred_element_type=jnp.float32)
        + bc_ref[...])

  return pl.pallas_call(
      body,
      grid=(n_acc // BN,),
      in_specs=[
          pl.BlockSpec((NC, BN, hh), lambda i: (0, i, 0)),
          pl.BlockSpec((NC, BN, hh), lambda i: (0, i, 0)),
          pl.BlockSpec((BN,), lambda i: (i,)),
          pl.BlockSpec((h_dim, h_dim), lambda i: (0, 0)),
          pl.BlockSpec((h_dim, h_dim), lambda i: (0, 0)),
          pl.BlockSpec((h_dim,), lambda i: (0,)),
          pl.BlockSpec((h_dim, oc), lambda i: (0, 0)),
          pl.BlockSpec((oc,), lambda i: (0,)),
      ],
      out_specs=pl.BlockSpec((BN, oc), lambda i: (i, 0)),
      out_shape=jax.ShapeDtypeStruct((n_acc, oc), jnp.float32),
  )(parts2, h1, invd, wl2, wr2, b2, wc_pad, bc_pad)


def _spread_fill(k, base, mod):
  return base + (jnp.arange(k, dtype=jnp.int32) % mod)


def _chunked_idx(vals, base, mod, n_rows, p, p_pad):
  cap = n_rows * p * CHUNK
  v = jnp.concatenate([vals, _spread_fill(cap - vals.shape[0], base, mod)])
  v = v.reshape(n_rows, p, CHUNK)
  padc = _spread_fill(n_rows * (p_pad + G - p) * CHUNK, base, mod).reshape(
      n_rows, p_pad + G - p, CHUNK)
  return jnp.concatenate([v, padc], axis=1)


def kernel(x, edge_index, W_l1, W_r1, b1, W_l2, W_r2, b2, Wc, bc):
  n, in_dim = x.shape
  e = edge_index.shape[1]
  h_dim = W_l1.shape[1]
  out_dim = Wc.shape[1]
  hh = h_dim // 2

  n_acc = -(-(n + 1) // (NS * 128)) * (NS * 128)
  rows_per_tile = n_acc // NS

  src = edge_index[0]
  dst = edge_index[1]

  p1 = -(-e // (NC * NS * CHUNK))
  p1_pad = -(-p1 // G) * G
  src1 = _chunked_idx(src, 0, n, NC * NS, p1, p1_pad).reshape(
      NC, NS, p1_pad + G, CHUNK)
  dst1 = _chunked_idx(dst, n, n_acc - n, NC * NS, p1, p1_pad).reshape(
      NC, NS, p1_pad + G, CHUNK)
  zeros1 = jnp.zeros((rows_per_tile, in_dim), jnp.float32)

  agg1 = _make_sc_agg(in_dim, p1_pad, n_acc)
  parts1 = agg1(x, src1, dst1, zeros1)

  zdeg = jnp.zeros((DCOL * n_acc,), jnp.float32)
  dst1f = dst1.reshape(NC, NS, (p1_pad + G) * CHUNK)
  deg_k = _make_sc_deg((p1_pad + G) * CHUNK, p1_pad, n_acc)
  degp = deg_k(dst1f, zdeg)

  x_pad = jnp.concatenate(
      [x, jnp.zeros((n_acc - n, in_dim), jnp.float32)], axis=0)
  h1, invd = _tc_layer1(parts1, degp, x_pad, W_l1, W_r1, b1,
                        n_acc, in_dim, h_dim)

  table2 = h1.reshape(NC * n_acc, hh)
  p2 = -(-e // (NS * CHUNK))
  p2_pad = -(-p2 // G) * G
  src2a = _chunked_idx(src, 0, n, NS, p2, p2_pad)
  src2 = jnp.stack([src2a, src2a + n_acc])
  dst2a = _chunked_idx(dst, n, n_acc - n, NS, p2, p2_pad)
  dst2 = jnp.stack([dst2a, dst2a])
  zeros2 = jnp.zeros((rows_per_tile, hh), jnp.float32)

  agg2 = _make_sc_agg(hh, p2_pad, n_acc)
  parts2 = agg2(table2, src2, dst2, zeros2)

  oc = 128
  wc_pad = jnp.zeros((h_dim, oc), jnp.float32).at[:, :out_dim].set(Wc)
  bc_pad = jnp.zeros((oc,), jnp.float32).at[:out_dim].set(bc)
  out = _tc_layer2(parts2, h1, invd, W_l2, W_r2, b2, wc_pad, bc_pad,
                   n_acc, h_dim)
  return out[:n, :out_dim]


# speedup vs baseline: 3.0350x; 1.0027x over previous
import functools

import jax
import jax.numpy as jnp
from jax import lax
from jax.experimental import pallas as pl
from jax.experimental.pallas import tpu as pltpu
from jax.experimental.pallas import tpu_sc as plsc

NC = 2
NS = 16
C1 = 96
G1 = 8
C2 = 128
G2 = 16
DCOL = 2


def _make_sc_agg(feat, p_pad, n_acc, chunk, g, idx4, want_deg):
  n_groups = p_pad // g
  pg = p_pad + g
  rows_per_tile = n_acc // NS
  mesh = plsc.VectorSubcoreMesh(core_axis_name="c", subcore_axis_name="s")

  out_type = [jax.ShapeDtypeStruct((NC, n_acc, feat), jnp.float32)]
  scratch = [
      pltpu.VMEM((2, g, chunk), jnp.int32),
      pltpu.VMEM((2, g, chunk), jnp.int32),
      pltpu.VMEM((chunk, feat), jnp.float32),
      pltpu.VMEM((chunk, feat), jnp.float32),
      pltpu.VMEM_SHARED((n_acc, feat), jnp.float32),
      pltpu.SemaphoreType.DMA,
      pltpu.SemaphoreType.DMA,
      pltpu.SemaphoreType.DMA,
  ]
  if want_deg:
    out_type.append(jax.ShapeDtypeStruct((NC, NS, n_acc), jnp.float32))
    scratch.append(pltpu.VMEM((DCOL * n_acc,), jnp.float32))

  @functools.partial(
      pl.kernel, out_type=tuple(out_type), mesh=mesh, scratch_types=scratch,
      compiler_params=pltpu.CompilerParams(needs_layout_passes=False))
  def k(*refs):
    if want_deg:
      (table_hbm, src_hbm, dst_hbm, zeros_hbm, zdeg_hbm, out_hbm, odeg_hbm,
       sbuf, dbuf, b0, b1, acc_sh, sg0, sg1, si, deg_v) = refs
    else:
      (table_hbm, src_hbm, dst_hbm, zeros_hbm, out_hbm,
       sbuf, dbuf, b0, b1, acc_sh, sg0, sg1, si) = refs

    c = lax.axis_index("c")
    s = lax.axis_index("s")
    bufs = (b0, b1)
    sems = (sg0, sg1)
    row0 = s * rows_per_tile
    table = table_hbm if idx4 else table_hbm.at[c]

    def idx_slice(arr, lo, size):
      return arr.at[c, s, pl.ds(lo, size)] if idx4 else \
          arr.at[s, pl.ds(lo, size)]

    pltpu.sync_copy(zeros_hbm, acc_sh.at[pl.ds(row0, rows_per_tile)])
    if want_deg:
      pltpu.sync_copy(zdeg_hbm, deg_v)
      lane = lax.iota(jnp.int32, 16)
      colbase = (lane & (DCOL - 1)) * n_acc
      ones16 = jnp.full((16,), 1.0, jnp.float32)
      gmasks = [(lane // DCOL) == q for q in range(16 // DCOL)]
    plsc.subcore_barrier()

    pltpu.sync_copy(idx_slice(src_hbm, 0, g), sbuf.at[0])
    pltpu.sync_copy(idx_slice(dst_hbm, 0, g), dbuf.at[0])
    pltpu.async_copy(table.at[sbuf.at[0, 0]], b0, sg0)

    def group(gi, carry):
      par = gi & 1
      nxt = 1 - par
      pltpu.async_copy(idx_slice(src_hbm, (gi + 1) * g, g), sbuf.at[nxt], si)
      pltpu.async_copy(idx_slice(dst_hbm, (gi + 1) * g, g), dbuf.at[nxt], si)
      for j in range(g):
        pb = j & 1
        nb = (j + 1) & 1
        if j + 1 < g:
          nxt_idx = sbuf.at[par, j + 1]
        else:
          pltpu.make_async_copy(idx_slice(src_hbm, 0, g), sbuf.at[par],
                                si).wait()
          pltpu.make_async_copy(idx_slice(dst_hbm, 0, g), dbuf.at[par],
                                si).wait()
          nxt_idx = sbuf.at[nxt, 0]
        pltpu.async_copy(table.at[nxt_idx], bufs[nb], sems[nb])
        pltpu.make_async_copy(table.at[pl.ds(0, chunk)],
                              bufs[pb], sems[pb]).wait()
        pltpu.sync_copy(bufs[pb], acc_sh.at[dbuf.at[par, j]], add=True)
        if want_deg:
          for q in range(chunk // 16):
            dv = dbuf[par, j, pl.ds(q * 16, 16)]
            idxv = dv + colbase
            for m in gmasks:
              plsc.addupdate_scatter(deg_v, [idxv], ones16, mask=m)
      return carry

    lax.fori_loop(0, n_groups, group, 0)
    pltpu.make_async_copy(table.at[pl.ds(0, chunk)], b0, sg0).wait()
    plsc.subcore_barrier()

    pltpu.sync_copy(acc_sh.at[pl.ds(row0, rows_per_tile)],
                    out_hbm.at[c, pl.ds(row0, rows_per_tile)])

    if want_deg:
      def red(i, carry):
        for u in range(4):
          o = (i * 4 + u) * 16
          v = deg_v[pl.ds(o, 16)]
          for d in range(1, DCOL):
            v += deg_v[pl.ds(d * n_acc + o, 16)]
          deg_v[pl.ds(o, 16)] = v
        return carry
      lax.fori_loop(0, n_acc // 64, red, 0)
      pltpu.sync_copy(deg_v.at[pl.ds(0, n_acc)], odeg_hbm.at[c, s])

  return k


def _tc_layer1(parts, degp, x_pad, wl, wr, b, n_acc, in_dim, h_dim):
  BN = 512
  feat = parts.shape[2]
  hh = h_dim // 2

  def body(parts_ref, degp_ref, x_ref, wl_ref, wr_ref, b_ref,
           h1_ref, invd_ref):
    deg = jnp.sum(degp_ref[...], axis=(0, 1))
    invd = 1.0 / jnp.maximum(deg, 1.0)
    p = parts_ref[0] + parts_ref[1]
    mean = p * invd[:, None]
    h = jnp.dot(mean, wl_ref[...], preferred_element_type=jnp.float32)
    h += jnp.dot(x_ref[...], wr_ref[...], preferred_element_type=jnp.float32)
    h = jnp.maximum(h + b_ref[...], 0.0)
    h1_ref[0] = h[:, :hh]
    h1_ref[1] = h[:, hh:]
    invd_ref[...] = invd

  return pl.pallas_call(
      body,
      grid=(n_acc // BN,),
      in_specs=[
          pl.BlockSpec((NC, BN, feat), lambda i: (0, i, 0)),
          pl.BlockSpec((NC, NS, BN), lambda i: (0, 0, i)),
          pl.BlockSpec((BN, in_dim), lambda i: (i, 0)),
          pl.BlockSpec((in_dim, h_dim), lambda i: (0, 0)),
          pl.BlockSpec((in_dim, h_dim), lambda i: (0, 0)),
          pl.BlockSpec((h_dim,), lambda i: (0,)),
      ],
      out_specs=[
          pl.BlockSpec((NC, BN, hh), lambda i: (0, i, 0)),
          pl.BlockSpec((BN,), lambda i: (i,)),
      ],
      out_shape=[
          jax.ShapeDtypeStruct((NC, n_acc, hh), jnp.float32),
          jax.ShapeDtypeStruct((n_acc,), jnp.float32),
      ],
  )(parts, degp, x_pad, wl, wr, b)


def _tc_layer2(parts2, h1, invd, wl2, wr2, b2, wc_pad, bc_pad, n_acc, h_dim):
  BN = 512
  hh = h_dim // 2
  oc = wc_pad.shape[1]

  def body(p2_ref, h1_ref, invd_ref, wl_ref, wr_ref, b_ref, wc_ref, bc_ref,
           out_ref):
    agg = jnp.concatenate([p2_ref[0], p2_ref[1]], axis=1)
    mean = agg * invd_ref[...][:, None]
    hp = jnp.concatenate([h1_ref[0], h1_ref[1]], axis=1)
    h = jnp.dot(mean, wl_ref[...], preferred_element_type=jnp.float32)
    h += jnp.dot(hp, wr_ref[...], preferred_element_type=jnp.float32)
    h = jnp.maximum(h + b_ref[...], 0.0)
    out_ref[...] = (
        jnp.dot(h, wc_ref[...], preferred_element_type=jnp.float32)
        + bc_ref[...])

  return pl.pallas_call(
      body,
      grid=(n_acc // BN,),
      in_specs=[
          pl.BlockSpec((NC, BN, hh), lambda i: (0, i, 0)),
          pl.BlockSpec((NC, BN, hh), lambda i: (0, i, 0)),
          pl.BlockSpec((BN,), lambda i: (i,)),
          pl.BlockSpec((h_dim, h_dim), lambda i: (0, 0)),
          pl.BlockSpec((h_dim, h_dim), lambda i: (0, 0)),
          pl.BlockSpec((h_dim,), lambda i: (0,)),
          pl.BlockSpec((h_dim, oc), lambda i: (0, 0)),
          pl.BlockSpec((oc,), lambda i: (0,)),
      ],
      out_specs=pl.BlockSpec((BN, oc), lambda i: (i, 0)),
      out_shape=jax.ShapeDtypeStruct((n_acc, oc), jnp.float32),
  )(parts2, h1, invd, wl2, wr2, b2, wc_pad, bc_pad)


def _spread_fill(k, base, mod):
  return base + (jnp.arange(k, dtype=jnp.int32) % mod)


def _chunked_idx(vals, base, mod, n_rows, p, p_pad, chunk, g):
  cap = n_rows * p * chunk
  v = jnp.concatenate([vals, _spread_fill(cap - vals.shape[0], base, mod)])
  v = v.reshape(n_rows, p, chunk)
  padc = _spread_fill(n_rows * (p_pad + g - p) * chunk, base, mod).reshape(
      n_rows, p_pad + g - p, chunk)
  return jnp.concatenate([v, padc], axis=1)


def kernel(x, edge_index, W_l1, W_r1, b1, W_l2, W_r2, b2, Wc, bc):
  n, in_dim = x.shape
  e = edge_index.shape[1]
  h_dim = W_l1.shape[1]
  out_dim = Wc.shape[1]
  hh = h_dim // 2

  n_acc = -(-(n + 1) // (NS * 128)) * (NS * 128)
  rows_per_tile = n_acc // NS

  src = edge_index[0]
  dst = edge_index[1]

  p1 = -(-e // (NC * NS * C1))
  p1_pad = -(-p1 // G1) * G1
  src1 = _chunked_idx(src, 0, n, NC * NS, p1, p1_pad, C1, G1).reshape(
      NC, NS, p1_pad + G1, C1)
  dst1 = _chunked_idx(dst, n, n_acc - n, NC * NS, p1, p1_pad, C1, G1).reshape(
      NC, NS, p1_pad + G1, C1)
  zeros1 = jnp.zeros((rows_per_tile, in_dim), jnp.float32)
  zdeg = jnp.zeros((DCOL * n_acc,), jnp.float32)

  agg1 = _make_sc_agg(in_dim, p1_pad, n_acc, C1, G1, idx4=True, want_deg=True)
  parts1, degp = agg1(x, src1, dst1, zeros1, zdeg)

  x_pad = jnp.concatenate(
      [x, jnp.zeros((n_acc - n, in_dim), jnp.float32)], axis=0)
  h1, invd = _tc_layer1(parts1, degp, x_pad, W_l1, W_r1, b1,
                        n_acc, in_dim, h_dim)

  p2 = -(-e // (NS * C2))
  p2_pad = -(-p2 // G2) * G2
  src2 = _chunked_idx(src, 0, n, NS, p2, p2_pad, C2, G2)
  dst2 = _chunked_idx(dst, n, n_acc - n, NS, p2, p2_pad, C2, G2)
  zeros2 = jnp.zeros((rows_per_tile, hh), jnp.float32)

  agg2 = _make_sc_agg(hh, p2_pad, n_acc, C2, G2, idx4=False, want_deg=False)
  (parts2,) = agg2(h1, src2, dst2, zeros2)

  oc = 128
  wc_pad = jnp.zeros((h_dim, oc), jnp.float32).at[:, :out_dim].set(Wc)
  bc_pad = jnp.zeros((oc,), jnp.float32).at[:out_dim].set(bc)
  out = _tc_layer2(parts2, h1, invd, W_l2, W_r2, b2, wc_pad, bc_pad,
                   n_acc, h_dim)
  return out[:n, :out_dim]


# speedup vs baseline: 3.1561x; 1.0399x over previous
import functools

import jax
import jax.numpy as jnp
from jax import lax
from jax.experimental import pallas as pl
from jax.experimental.pallas import tpu as pltpu
from jax.experimental.pallas import tpu_sc as plsc

NC = 2
NS = 16
C1 = 96
G1 = 4
C2 = 128
G2 = 16
DCOL = 2
BN = 2048


def _make_sc_agg(feat, ptc, n_acc, chunk, g, idx4, want_deg):
  n_groups = ptc // g
  rows_per_tile = n_acc // NS
  mesh = plsc.VectorSubcoreMesh(core_axis_name="c", subcore_axis_name="s")

  out_type = [jax.ShapeDtypeStruct((NC, n_acc, feat), jnp.float32)]
  scratch = [
      pltpu.VMEM((2, g, chunk), jnp.int32),
      pltpu.VMEM((2, g, chunk), jnp.int32),
      pltpu.VMEM((chunk, feat), jnp.float32),
      pltpu.VMEM((chunk, feat), jnp.float32),
      pltpu.VMEM_SHARED((n_acc, feat), jnp.float32),
      pltpu.SemaphoreType.DMA,
      pltpu.SemaphoreType.DMA,
      pltpu.SemaphoreType.DMA,
  ]
  if want_deg:
    out_type.append(jax.ShapeDtypeStruct((NC, NS, n_acc), jnp.float32))
    scratch.append(pltpu.VMEM((DCOL * n_acc,), jnp.float32))

  @functools.partial(
      pl.kernel, out_type=tuple(out_type), mesh=mesh, scratch_types=scratch,
      compiler_params=pltpu.CompilerParams(needs_layout_passes=False))
  def k(*refs):
    if want_deg:
      (table_hbm, src_hbm, dst_hbm, zeros_hbm, zdeg_hbm, out_hbm, odeg_hbm,
       sbuf, dbuf, b0, b1, acc_sh, sg0, sg1, si, deg_v) = refs
    else:
      (table_hbm, src_hbm, dst_hbm, zeros_hbm, out_hbm,
       sbuf, dbuf, b0, b1, acc_sh, sg0, sg1, si) = refs

    c = lax.axis_index("c")
    s = lax.axis_index("s")
    bufs = (b0, b1)
    sems = (sg0, sg1)
    row0 = s * rows_per_tile
    table = table_hbm if idx4 else table_hbm.at[c]
    bt = (c * NS + s) * ptc if idx4 else s * ptc

    pltpu.sync_copy(zeros_hbm, acc_sh.at[pl.ds(row0, rows_per_tile)])
    if want_deg:
      pltpu.sync_copy(zdeg_hbm, deg_v)
      lane = lax.iota(jnp.int32, 16)
      colbase = (lane & (DCOL - 1)) * n_acc
      ones16 = jnp.full((16,), 1.0, jnp.float32)
      gmasks = [(lane // DCOL) == q for q in range(16 // DCOL)]
    plsc.subcore_barrier()

    pltpu.sync_copy(src_hbm.at[pl.ds(bt, g)], sbuf.at[0])
    pltpu.sync_copy(dst_hbm.at[pl.ds(bt, g)], dbuf.at[0])
    pltpu.async_copy(table.at[sbuf.at[0, 0]], b0, sg0)

    def group(gi, carry):
      par = gi & 1
      nxt = 1 - par
      nc0 = bt + (gi + 1) * g
      pltpu.async_copy(src_hbm.at[pl.ds(nc0, g)], sbuf.at[nxt], si)
      pltpu.async_copy(dst_hbm.at[pl.ds(nc0, g)], dbuf.at[nxt], si)
      for j in range(g):
        pb = j & 1
        nb = (j + 1) & 1
        if j + 1 < g:
          nxt_idx = sbuf.at[par, j + 1]
        else:
          pltpu.make_async_copy(src_hbm.at[pl.ds(0, g)],
                                sbuf.at[par], si).wait()
          pltpu.make_async_copy(dst_hbm.at[pl.ds(0, g)],
                                dbuf.at[par], si).wait()
          nxt_idx = sbuf.at[nxt, 0]
        pltpu.async_copy(table.at[nxt_idx], bufs[nb], sems[nb])
        pltpu.make_async_copy(table.at[pl.ds(0, chunk)],
                              bufs[pb], sems[pb]).wait()
        pltpu.sync_copy(bufs[pb], acc_sh.at[dbuf.at[par, j]], add=True)
        if want_deg:
          for q in range(chunk // 16):
            dv = dbuf[par, j, pl.ds(q * 16, 16)]
            idxv = dv + colbase
            for m in gmasks:
              plsc.addupdate_scatter(deg_v, [idxv], ones16, mask=m)
      return carry

    lax.fori_loop(0, n_groups, group, 0)
    pltpu.make_async_copy(table.at[pl.ds(0, chunk)], b0, sg0).wait()
    plsc.subcore_barrier()

    pltpu.sync_copy(acc_sh.at[pl.ds(row0, rows_per_tile)],
                    out_hbm.at[c, pl.ds(row0, rows_per_tile)])

    if want_deg:
      def red(i, carry):
        for u in range(4):
          o = (i * 4 + u) * 16
          v = deg_v[pl.ds(o, 16)]
          for d in range(1, DCOL):
            v += deg_v[pl.ds(d * n_acc + o, 16)]
          deg_v[pl.ds(o, 16)] = v
        return carry
      lax.fori_loop(0, n_acc // 64, red, 0)
      pltpu.sync_copy(deg_v.at[pl.ds(0, n_acc)], odeg_hbm.at[c, s])

  return k


def _tc_lin(inp, w, b, n_acc, split_in):
  kdim, m = w.shape

  def body(in_ref, w_ref, b_ref, out_ref):
    if split_in:
      a = jnp.concatenate([in_ref[0], in_ref[1]], axis=1)
    else:
      a = in_ref[...]
    out_ref[...] = (
        jnp.dot(a, w_ref[...], preferred_element_type=jnp.float32)
        + b_ref[...])

  in_spec = (pl.BlockSpec((NC, BN, kdim // 2), lambda i: (0, i, 0))
             if split_in else pl.BlockSpec((BN, kdim), lambda i: (i, 0)))
  return pl.pallas_call(
      body,
      grid=(n_acc // BN,),
      in_specs=[
          in_spec,
          pl.BlockSpec((kdim, m), lambda i: (0, 0)),
          pl.BlockSpec((m,), lambda i: (0,)),
      ],
      out_specs=pl.BlockSpec((BN, m), lambda i: (i, 0)),
      out_shape=jax.ShapeDtypeStruct((n_acc, m), jnp.float32),
  )(inp, w, b)


def _tc_layer1(parts, degp, xr, wl, n_acc, in_dim, h_dim):
  feat = parts.shape[2]
  hh = h_dim // 2

  def body(parts_ref, degp_ref, xr_ref, wl_ref, h1_ref, invd_ref):
    deg = jnp.sum(degp_ref[...], axis=(0, 1))
    invd = 1.0 / jnp.maximum(deg, 1.0)
    p = parts_ref[0] + parts_ref[1]
    mean = p * invd[:, None]
    h = jnp.dot(mean, wl_ref[...], preferred_element_type=jnp.float32)
    h = jnp.maximum(h + xr_ref[...], 0.0)
    h1_ref[0] = h[:, :hh]
    h1_ref[1] = h[:, hh:]
    invd_ref[...] = invd

  return pl.pallas_call(
      body,
      grid=(n_acc // BN,),
      in_specs=[
          pl.BlockSpec((NC, BN, feat), lambda i: (0, i, 0)),
          pl.BlockSpec((NC, NS, BN), lambda i: (0, 0, i)),
          pl.BlockSpec((BN, h_dim), lambda i: (i, 0)),
          pl.BlockSpec((in_dim, h_dim), lambda i: (0, 0)),
      ],
      out_specs=[
          pl.BlockSpec((NC, BN, hh), lambda i: (0, i, 0)),
          pl.BlockSpec((BN,), lambda i: (i,)),
      ],
      out_shape=[
          jax.ShapeDtypeStruct((NC, n_acc, hh), jnp.float32),
          jax.ShapeDtypeStruct((n_acc,), jnp.float32),
      ],
  )(parts, degp, xr, wl)


def _tc_layer2(parts2, invd, hr, wl2, wc_pad, bc_pad, n_acc, h_dim):
  hh = h_dim // 2
  oc = wc_pad.shape[1]

  def body(p2_ref, invd_ref, hr_ref, wl_ref, wc_ref, bc_ref, out_ref):
    agg = jnp.concatenate([p2_ref[0], p2_ref[1]], axis=1)
    mean = agg * invd_ref[...][:, None]
    h = jnp.dot(mean, wl_ref[...], preferred_element_type=jnp.float32)
    h = jnp.maximum(h + hr_ref[...], 0.0)
    out_ref[...] = (
        jnp.dot(h, wc_ref[...], preferred_element_type=jnp.float32)
        + bc_ref[...])

  return pl.pallas_call(
      body,
      grid=(n_acc // BN,),
      in_specs=[
          pl.BlockSpec((NC, BN, hh), lambda i: (0, i, 0)),
          pl.BlockSpec((BN,), lambda i: (i,)),
          pl.BlockSpec((BN, h_dim), lambda i: (i, 0)),
          pl.BlockSpec((h_dim, h_dim), lambda i: (0, 0)),
          pl.BlockSpec((h_dim, oc), lambda i: (0, 0)),
          pl.BlockSpec((oc,), lambda i: (0,)),
      ],
      out_specs=pl.BlockSpec((BN, oc), lambda i: (i, 0)),
      out_shape=jax.ShapeDtypeStruct((n_acc, oc), jnp.float32),
  )(parts2, invd, hr, wl2, wc_pad, bc_pad)


def _spread_fill(k, base, mod):
  return base + (jnp.arange(k, dtype=jnp.int32) % mod)


def kernel(x, edge_index, W_l1, W_r1, b1, W_l2, W_r2, b2, Wc, bc):
  n, in_dim = x.shape
  e = edge_index.shape[1]
  h_dim = W_l1.shape[1]
  out_dim = Wc.shape[1]
  hh = h_dim // 2

  n_acc = -(-(n + 1) // (NS * 128)) * (NS * 128)
  rows_per_tile = n_acc // NS

  src = edge_index[0]
  dst = edge_index[1]

  def flat_idx(vals, base, mod, tiles, ptc, chunk, g):
    total = (tiles * ptc + g) * chunk
    return jnp.concatenate(
        [vals, _spread_fill(total - vals.shape[0], base, mod)])

  def round_up(a, m):
    return -(-a // m) * m

  ptc1 = round_up(-(-e // (NC * NS * C1)), G1)
  src1 = flat_idx(src, 0, n, NC * NS, ptc1, C1, G1).reshape(-1, C1)
  dst1 = flat_idx(dst, n, n_acc - n, NC * NS, ptc1, C1, G1).reshape(-1, C1)
  zeros1 = jnp.zeros((rows_per_tile, in_dim), jnp.float32)
  zdeg = jnp.zeros((DCOL * n_acc,), jnp.float32)

  agg1 = _make_sc_agg(in_dim, ptc1, n_acc, C1, G1, idx4=True, want_deg=True)
  parts1, degp = agg1(x, src1, dst1, zeros1, zdeg)

  x_pad = jnp.concatenate(
      [x, jnp.zeros((n_acc - n, in_dim), jnp.float32)], axis=0)
  xr = _tc_lin(x_pad, W_r1, b1, n_acc, split_in=False)

  h1, invd = _tc_layer1(parts1, degp, xr, W_l1, n_acc, in_dim, h_dim)

  ptc2 = round_up(-(-e // (NS * C2)), G2)
  src2 = flat_idx(src, 0, n, NS, ptc2, C2, G2).reshape(-1, C2)
  dst2 = flat_idx(dst, n, n_acc - n, NS, ptc2, C2, G2).reshape(-1, C2)
  zeros2 = jnp.zeros((rows_per_tile, hh), jnp.float32)

  agg2 = _make_sc_agg(hh, ptc2, n_acc, C2, G2, idx4=False, want_deg=False)
  (parts2,) = agg2(h1, src2, dst2, zeros2)

  hr = _tc_lin(h1, W_r2, b2, n_acc, split_in=True)

  oc = 128
  wc_pad = jnp.zeros((h_dim, oc), jnp.float32).at[:, :out_dim].set(Wc)
  bc_pad = jnp.zeros((oc,), jnp.float32).at[:out_dim].set(bc)
  out = _tc_layer2(parts2, invd, hr, W_l2, wc_pad, bc_pad, n_acc, h_dim)
  return out[:n, :out_dim]


# speedup vs baseline: 3.2767x; 1.0382x over previous
import functools

import jax
import jax.numpy as jnp
from jax import lax
from jax.experimental import pallas as pl
from jax.experimental.pallas import tpu as pltpu
from jax.experimental.pallas import tpu_sc as plsc

NC = 2
NS = 16
C1 = 96
G1 = 2
C2 = 128
G2 = 8
DCOL = 2
BN = 2048


def _make_sc_agg(feat, ptc, n_acc, chunk, g, idx4, want_deg):
  n_groups = ptc // g
  rows_per_tile = n_acc // NS
  mesh = plsc.VectorSubcoreMesh(core_axis_name="c", subcore_axis_name="s")

  out_type = [jax.ShapeDtypeStruct((NC, n_acc, feat), jnp.float32)]
  scratch = [
      pltpu.VMEM((2, g, chunk), jnp.int32),
      pltpu.VMEM((2, g, chunk), jnp.int32),
      pltpu.VMEM((chunk, feat), jnp.float32),
      pltpu.VMEM((chunk, feat), jnp.float32),
      pltpu.VMEM_SHARED((n_acc, feat), jnp.float32),
      pltpu.SemaphoreType.DMA,
      pltpu.SemaphoreType.DMA,
      pltpu.SemaphoreType.DMA,
  ]
  if want_deg:
    out_type.append(jax.ShapeDtypeStruct((NC, NS, n_acc), jnp.float32))
    scratch.append(pltpu.VMEM((DCOL * n_acc,), jnp.float32))

  @functools.partial(
      pl.kernel, out_type=tuple(out_type), mesh=mesh, scratch_types=scratch,
      compiler_params=pltpu.CompilerParams(needs_layout_passes=False))
  def k(*refs):
    if want_deg:
      (table_hbm, ei_hbm, zeros_hbm, zdeg_hbm, out_hbm, odeg_hbm,
       sbuf, dbuf, b0, b1, acc_sh, sg0, sg1, si, deg_v) = refs
    else:
      (table_hbm, ei_hbm, zeros_hbm, out_hbm,
       sbuf, dbuf, b0, b1, acc_sh, sg0, sg1, si) = refs
    src_hbm = ei_hbm.at[0]
    dst_hbm = ei_hbm.at[1]

    c = lax.axis_index("c")
    s = lax.axis_index("s")
    bufs = (b0, b1)
    sems = (sg0, sg1)
    row0 = s * rows_per_tile
    table = table_hbm if idx4 else table_hbm.at[c]
    bt = (c * NS + s) * ptc if idx4 else s * ptc

    pltpu.sync_copy(zeros_hbm, acc_sh.at[pl.ds(row0, rows_per_tile)])
    if want_deg:
      pltpu.sync_copy(zdeg_hbm, deg_v)
      lane = lax.iota(jnp.int32, 16)
      colbase = (lane & (DCOL - 1)) * n_acc
      ones16 = jnp.full((16,), 1.0, jnp.float32)
      gmasks = [(lane // DCOL) == q for q in range(16 // DCOL)]
    plsc.subcore_barrier()

    pltpu.sync_copy(src_hbm.at[pl.ds(bt, g)], sbuf.at[0])
    pltpu.sync_copy(dst_hbm.at[pl.ds(bt, g)], dbuf.at[0])
    pltpu.async_copy(table.at[sbuf.at[0, 0]], b0, sg0)

    def group(gi, carry):
      par = gi & 1
      nxt = 1 - par
      nc0 = bt + (gi + 1) * g
      pltpu.async_copy(src_hbm.at[pl.ds(nc0, g)], sbuf.at[nxt], si)
      pltpu.async_copy(dst_hbm.at[pl.ds(nc0, g)], dbuf.at[nxt], si)
      for j in range(g):
        pb = j & 1
        nb = (j + 1) & 1
        if j + 1 < g:
          nxt_idx = sbuf.at[par, j + 1]
        else:
          pltpu.make_async_copy(src_hbm.at[pl.ds(0, g)],
                                sbuf.at[par], si).wait()
          pltpu.make_async_copy(dst_hbm.at[pl.ds(0, g)],
                                dbuf.at[par], si).wait()
          nxt_idx = sbuf.at[nxt, 0]
        pltpu.async_copy(table.at[nxt_idx], bufs[nb], sems[nb])
        pltpu.make_async_copy(table.at[pl.ds(0, chunk)],
                              bufs[pb], sems[pb]).wait()
        pltpu.sync_copy(bufs[pb], acc_sh.at[dbuf.at[par, j]], add=True)
        if want_deg:
          for q in range(chunk // 16):
            dv = dbuf[par, j, pl.ds(q * 16, 16)]
            idxv = dv + colbase
            for m in gmasks:
              plsc.addupdate_scatter(deg_v, [idxv], ones16, mask=m)
      return carry

    lax.fori_loop(0, n_groups, group, 0)
    pltpu.make_async_copy(table.at[pl.ds(0, chunk)], b0, sg0).wait()
    plsc.subcore_barrier()

    pltpu.sync_copy(acc_sh.at[pl.ds(row0, rows_per_tile)],
                    out_hbm.at[c, pl.ds(row0, rows_per_tile)])

    if want_deg:
      def red(i, carry):
        for u in range(4):
          o = (i * 4 + u) * 16
          v = deg_v[pl.ds(o, 16)]
          for d in range(1, DCOL):
            v += deg_v[pl.ds(d * n_acc + o, 16)]
          deg_v[pl.ds(o, 16)] = v
        return carry
      lax.fori_loop(0, n_acc // 64, red, 0)
      pltpu.sync_copy(deg_v.at[pl.ds(0, n_acc)], odeg_hbm.at[c, s])

  return k


def _tc_lin(inp, w, b, n_acc, split_in):
  kdim, m = w.shape

  def body(in_ref, w_ref, b_ref, out_ref):
    if split_in:
      a = jnp.concatenate([in_ref[0], in_ref[1]], axis=1)
    else:
      a = in_ref[...]
    out_ref[...] = (
        jnp.dot(a, w_ref[...], preferred_element_type=jnp.float32)
        + b_ref[...]).astype(jnp.bfloat16)

  in_spec = (pl.BlockSpec((NC, BN, kdim // 2), lambda i: (0, i, 0))
             if split_in else pl.BlockSpec((BN, kdim), lambda i: (i, 0)))
  return pl.pallas_call(
      body,
      grid=(n_acc // BN,),
      in_specs=[
          in_spec,
          pl.BlockSpec((kdim, m), lambda i: (0, 0)),
          pl.BlockSpec((m,), lambda i: (0,)),
      ],
      out_specs=pl.BlockSpec((BN, m), lambda i: (i, 0)),
      out_shape=jax.ShapeDtypeStruct((n_acc, m), jnp.bfloat16),
  )(inp, w, b)


def _tc_layer1(parts, degp, xr, wl, n_acc, in_dim, h_dim):
  feat = parts.shape[2]
  hh = h_dim // 2

  def body(parts_ref, degp_ref, xr_ref, wl_ref, h1_ref, invd_ref):
    deg = jnp.sum(degp_ref[...], axis=(0, 1))
    invd = 1.0 / jnp.maximum(deg, 1.0)
    p = parts_ref[0] + parts_ref[1]
    mean = p * invd[:, None]
    h = jnp.dot(mean, wl_ref[...], preferred_element_type=jnp.float32)
    h = jnp.maximum(h + xr_ref[...].astype(jnp.float32), 0.0)
    h1_ref[0] = h[:, :hh]
    h1_ref[1] = h[:, hh:]
    invd_ref[...] = invd

  return pl.pallas_call(
      body,
      grid=(1,),
      in_specs=[
          pl.BlockSpec((NC, n_acc, feat), lambda i: (0, 0, 0)),
          pl.BlockSpec((NC, NS, n_acc), lambda i: (0, 0, 0)),
          pl.BlockSpec((n_acc, h_dim), lambda i: (0, 0)),
          pl.BlockSpec((in_dim, h_dim), lambda i: (0, 0)),
      ],
      out_specs=[
          pl.BlockSpec((NC, n_acc, hh), lambda i: (0, 0, 0)),
          pl.BlockSpec((n_acc,), lambda i: (0,)),
      ],
      out_shape=[
          jax.ShapeDtypeStruct((NC, n_acc, hh), jnp.float32),
          jax.ShapeDtypeStruct((n_acc,), jnp.float32),
      ],
  )(parts, degp, xr, wl)


def _tc_layer2(parts2, invd, hr, wl2, wc_pad, bc_pad, n_acc, h_dim):
  hh = h_dim // 2
  oc = wc_pad.shape[1]

  def body(p2_ref, invd_ref, hr_ref, wl_ref, wc_ref, bc_ref, out_ref):
    agg = jnp.concatenate([p2_ref[0], p2_ref[1]], axis=1)
    mean = agg * invd_ref[...][:, None]
    h = jnp.dot(mean, wl_ref[...], preferred_element_type=jnp.float32)
    h = jnp.maximum(h + hr_ref[...].astype(jnp.float32), 0.0)
    out_ref[...] = (
        jnp.dot(h, wc_ref[...], preferred_element_type=jnp.float32)
        + bc_ref[...])

  return pl.pallas_call(
      body,
      grid=(1,),
      in_specs=[
          pl.BlockSpec((NC, n_acc, hh), lambda i: (0, 0, 0)),
          pl.BlockSpec((n_acc,), lambda i: (0,)),
          pl.BlockSpec((n_acc, h_dim), lambda i: (0, 0)),
          pl.BlockSpec((h_dim, h_dim), lambda i: (0, 0)),
          pl.BlockSpec((h_dim, oc), lambda i: (0, 0)),
          pl.BlockSpec((oc,), lambda i: (0,)),
      ],
      out_specs=pl.BlockSpec((n_acc, oc), lambda i: (0, 0)),
      out_shape=jax.ShapeDtypeStruct((n_acc, oc), jnp.float32),
  )(parts2, invd, hr, wl2, wc_pad, bc_pad)


def _spread_fill(k, base, mod):
  return base + (jnp.arange(k, dtype=jnp.int32) % mod)


def kernel(x, edge_index, W_l1, W_r1, b1, W_l2, W_r2, b2, Wc, bc):
  n, in_dim = x.shape
  e = edge_index.shape[1]
  h_dim = W_l1.shape[1]
  out_dim = Wc.shape[1]
  hh = h_dim // 2

  n_acc = -(-(n + 1) // (NS * 128)) * (NS * 128)
  rows_per_tile = n_acc // NS

  def padded_ei(tiles, ptc, chunk, g):
    total = (tiles * ptc + g) * chunk
    fills = jnp.stack([_spread_fill(total - e, 0, n),
                       _spread_fill(total - e, n, n_acc - n)])
    return jnp.concatenate([edge_index, fills], axis=1).reshape(
        2, -1, chunk)

  def round_up(a, m):
    return -(-a // m) * m

  ptc1 = round_up(-(-e // (NC * NS * C1)), G1)
  ei1 = padded_ei(NC * NS, ptc1, C1, G1)
  zeros1 = jnp.zeros((rows_per_tile, in_dim), jnp.float32)
  zdeg = jnp.zeros((DCOL * n_acc,), jnp.float32)

  agg1 = _make_sc_agg(in_dim, ptc1, n_acc, C1, G1, idx4=True, want_deg=True)
  parts1, degp = agg1(x, ei1, zeros1, zdeg)

  x_pad = jnp.concatenate(
      [x, jnp.zeros((n_acc - n, in_dim), jnp.float32)], axis=0)
  xr = _tc_lin(x_pad, W_r1, b1, n_acc, split_in=False)

  h1, invd = _tc_layer1(parts1, degp, xr, W_l1, n_acc, in_dim, h_dim)

  ptc2 = round_up(-(-e // (NS * C2)), G2)
  ei2 = padded_ei(NS, ptc2, C2, G2)
  zeros2 = jnp.zeros((rows_per_tile, hh), jnp.float32)

  agg2 = _make_sc_agg(hh, ptc2, n_acc, C2, G2, idx4=False, want_deg=False)
  (parts2,) = agg2(h1, ei2, zeros2)

  hr = _tc_lin(h1, W_r2, b2, n_acc, split_in=True)

  oc = 128
  wc_pad = jnp.zeros((h_dim, oc), jnp.float32).at[:, :out_dim].set(Wc)
  bc_pad = jnp.zeros((oc,), jnp.float32).at[:out_dim].set(bc)
  out = _tc_layer2(parts2, invd, hr, W_l2, wc_pad, bc_pad, n_acc, h_dim)
  return out[:n, :out_dim]
